# baseline (speedup 1.0000x reference)
import functools

import jax
import jax.numpy as jnp
from jax import lax
from jax.experimental import pallas as pl
from jax.experimental.pallas import tpu as pltpu
from jax.experimental.pallas import tpu_sc as plsc

NC = 2
NS = 16
NW = NC * NS
N = 10000
D = 128
E = 320000
RPT = 624
TAIL = NS * RPT
CNTW = 16

CH1, KB1, DEPTH1 = 128, 6, 2
CH2, KB2, DEPTH2 = 128, 13, 2


def _sc_common(with_cnt, ch, kb, depth,
               x_hbm, src_hbm, dst_hbm, zeros_hbm, zcnt_hbm, aggp, cntp,
               agg_sh, cnt_sh, srcb, dstb, rows, ones, sem):
    c = lax.axis_index("c")
    s = lax.axis_index("s")
    wid = c * NS + s
    start = s * RPT

    if with_cnt:
        one16 = jnp.ones((16,), jnp.float32)

        def orow(r, _):
            ones[r, :] = one16
            return 0
        lax.fori_loop(0, ch, orow, 0)

    pltpu.sync_copy(zeros_hbm.at[pl.ds(start, RPT)],
                    agg_sh.at[pl.ds(start, RPT)])
    if with_cnt:
        pltpu.sync_copy(zcnt_hbm.at[pl.ds(start, RPT)],
                        cnt_sh.at[pl.ds(start, RPT)])

    @pl.when(s == NS - 1)
    def _ztail():
        pltpu.sync_copy(zeros_hbm.at[pl.ds(TAIL, N - TAIL)],
                        agg_sh.at[pl.ds(TAIL, N - TAIL)])
        if with_cnt:
            pltpu.sync_copy(zcnt_hbm.at[pl.ds(TAIL, N - TAIL)],
                            cnt_sh.at[pl.ds(TAIL, N - TAIL)])

    plsc.subcore_barrier()

    ntotal = E // ch
    base = ntotal // NW
    rem = ntotal % NW
    cw = base + jnp.where(wid < rem, 1, 0)
    rbase = wid * base + jnp.minimum(wid, rem)

    def load_bulk(b):
        bb = b % 2
        pltpu.sync_copy(src_hbm.at[pl.ds(rbase + b * kb, kb)], srcb.at[bb])
        pltpu.sync_copy(dst_hbm.at[pl.ds(rbase + b * kb, kb)], dstb.at[bb])

    def fire(j):
        bb = (j // kb) % 2
        pltpu.async_copy(x_hbm.at[srcb.at[bb, j % kb]], rows.at[j % depth],
                         sem.at[j % depth])

    load_bulk(0)
    fire(0)
    for k in range(1, depth - 1):
        @pl.when(k < cw)
        def _prefire(k=k):
            fire(k)

    def chunk(j, _):
        nj = j + depth - 1

        @pl.when(jnp.logical_and(nj % kb == 0, nj < cw))
        def _load():
            load_bulk(nj // kb)

        @pl.when(nj < cw)
        def _fire():
            fire(nj)

        bb = (j // kb) % 2
        pltpu.make_async_copy(x_hbm.at[pl.ds(0, ch)], rows.at[j % depth],
                              sem.at[j % depth]).wait()
        pltpu.sync_copy(rows.at[j % depth], agg_sh.at[dstb.at[bb, j % kb]],
                        add=True)
        if with_cnt:
            pltpu.sync_copy(ones, cnt_sh.at[dstb.at[bb, j % kb]], add=True)
        return 0
    lax.fori_loop(0, cw, chunk, 0)

    plsc.subcore_barrier()

    pltpu.sync_copy(agg_sh.at[pl.ds(start, RPT)],
                    aggp.at[c, pl.ds(start, RPT)])
    if with_cnt:
        pltpu.sync_copy(cnt_sh.at[pl.ds(start, RPT)],
                        cntp.at[c, pl.ds(start, RPT)])

    @pl.when(s == NS - 1)
    def _ctail():
        pltpu.sync_copy(agg_sh.at[pl.ds(TAIL, N - TAIL)],
                        aggp.at[c, pl.ds(TAIL, N - TAIL)])
        if with_cnt:
            pltpu.sync_copy(cnt_sh.at[pl.ds(TAIL, N - TAIL)],
                            cntp.at[c, pl.ds(TAIL, N - TAIL)])


@functools.cache
def _sc_agg(with_cnt: bool, ch: int, kb: int, depth: int):
    mesh = plsc.VectorSubcoreMesh(core_axis_name="c", subcore_axis_name="s",
                                  num_cores=NC, num_subcores=NS)
    out_type = [jax.ShapeDtypeStruct((NC, N, D), jnp.float32)]
    scratch = [
        pltpu.VMEM_SHARED((N, D), jnp.float32),
    ]
    if with_cnt:
        out_type.append(jax.ShapeDtypeStruct((NC, N, CNTW), jnp.float32))
        scratch.append(pltpu.VMEM_SHARED((N, CNTW), jnp.float32))
    scratch += [
        pltpu.VMEM((2, kb, ch), jnp.int32),
        pltpu.VMEM((2, kb, ch), jnp.int32),
        pltpu.VMEM((depth, ch, D), jnp.float32),
    ]
    if with_cnt:
        scratch.append(pltpu.VMEM((ch, CNTW), jnp.float32))
    scratch.append(pltpu.SemaphoreType.DMA((depth,)))

    if with_cnt:
        def body(x_hbm, src_hbm, dst_hbm, zeros_hbm, zcnt_hbm, aggp, cntp,
                 agg_sh, cnt_sh, srcb, dstb, rows, ones, sem):
            _sc_common(True, ch, kb, depth,
                       x_hbm, src_hbm, dst_hbm, zeros_hbm, zcnt_hbm, aggp,
                       cntp, agg_sh, cnt_sh, srcb, dstb, rows, ones, sem)
    else:
        def body(x_hbm, src_hbm, dst_hbm, zeros_hbm, aggp,
                 agg_sh, srcb, dstb, rows, sem):
            _sc_common(False, ch, kb, depth,
                       x_hbm, src_hbm, dst_hbm, zeros_hbm, None, aggp, None,
                       agg_sh, None, srcb, dstb, rows, None, sem)

    return pl.kernel(body, out_type=tuple(out_type), mesh=mesh,
                     scratch_types=tuple(scratch),
                     compiler_params=pltpu.CompilerParams(
                         use_tc_tiling_on_sc=False))


def _dot_t(a, w):
    return lax.dot_general(a, w, (((1,), (1,)), ((), ())),
                           preferred_element_type=jnp.float32)


def _tc_xr_body(x_ref, wr_ref, o_ref):
    o_ref[...] = _dot_t(x_ref[...], wr_ref[...])


def _tc_layer_body(act, aggp_ref, cntp_ref, xr_ref, wl_ref, bl_ref, o_ref):
    agg = aggp_ref[0] + aggp_ref[1]
    cnt = cntp_ref[0, :, 0:1] + cntp_ref[1, :, 0:1]
    mean = agg / jnp.maximum(cnt, 1.0)
    out = _dot_t(mean, wl_ref[...]) + bl_ref[...] + xr_ref[...]
    if act == "elu":
        o_ref[...] = jnp.where(out > 0, out,
                               jnp.exp(jnp.minimum(out, 0.0)) - 1.0)
    else:
        m = jnp.max(out, axis=1, keepdims=True)
        lse = jnp.log(jnp.sum(jnp.exp(out - m), axis=1, keepdims=True)) + m
        o_ref[...] = out - lse


BR = 1000


@functools.cache
def _tc_xr():
    return pl.pallas_call(
        _tc_xr_body,
        grid=(N // BR,),
        in_specs=[
            pl.BlockSpec((BR, D), lambda i: (i, 0)),
            pl.BlockSpec((D, D), lambda i: (0, 0)),
        ],
        out_specs=pl.BlockSpec((BR, D), lambda i: (i, 0)),
        out_shape=jax.ShapeDtypeStruct((N, D), jnp.float32),
    )


@functools.cache
def _tc_layer(act: str):
    return pl.pallas_call(
        functools.partial(_tc_layer_body, act),
        grid=(N // BR,),
        in_specs=[
            pl.BlockSpec((NC, BR, D), lambda i: (0, i, 0)),
            pl.BlockSpec((NC, BR, CNTW), lambda i: (0, i, 0)),
            pl.BlockSpec((BR, D), lambda i: (i, 0)),
            pl.BlockSpec((D, D), lambda i: (0, 0)),
            pl.BlockSpec((1, D), lambda i: (0, 0)),
        ],
        out_specs=pl.BlockSpec((BR, D), lambda i: (i, 0)),
        out_shape=jax.ShapeDtypeStruct((N, D), jnp.float32),
    )


@jax.jit
def kernel(x, edge_index, W1l, b1l, W1r, W2l, b2l, W2r):
    src = edge_index[0].astype(jnp.int32)
    dst = edge_index[1].astype(jnp.int32)
    src1 = src.reshape(E // CH1, CH1)
    dst1 = dst.reshape(E // CH1, CH1)
    src2 = src.reshape(E // CH2, CH2)
    dst2 = dst.reshape(E // CH2, CH2)
    zeros = jnp.zeros((N, D), jnp.float32)
    zcnt = jnp.zeros((N, CNTW), jnp.float32)
    xr1 = _tc_xr()(x, W1r)
    aggp1, cntp = _sc_agg(True, CH1, KB1, DEPTH1)(x, src1, dst1, zeros, zcnt)
    h = _tc_layer("elu")(aggp1, cntp, xr1, W1l, b1l.reshape(1, D))
    xr2 = _tc_xr()(h, W2r)
    aggp2, = _sc_agg(False, CH2, KB2, DEPTH2)(h, src2, dst2, zeros)
    return _tc_layer("lsm")(aggp2, cntp, xr2, W2l, b2l.reshape(1, D))


# speedup vs baseline: 1.0335x; 1.0335x over previous
import functools

import jax
import jax.numpy as jnp
from jax import lax
from jax.experimental import pallas as pl
from jax.experimental.pallas import tpu as pltpu
from jax.experimental.pallas import tpu_sc as plsc

NC = 2
NS = 16
NW = NC * NS
N = 10000
D = 128
E = 320000
RPT = 624
TAIL = NS * RPT
CNTW = 16

CH1, KB1, DEPTH1 = 128, 6, 2
CH2, KB2, DEPTH2 = 128, 13, 2


def _sc_common(with_cnt, ch, kb, depth,
               x_hbm, src_hbm, dst_hbm, zeros_hbm, zcnt_hbm, aggp, cntp,
               agg_sh, cnt_sh, srcb, dstb, rows, ones, gsem, ssem):
    c = lax.axis_index("c")
    s = lax.axis_index("s")
    wid = c * NS + s
    start = s * RPT

    if with_cnt:
        one16 = jnp.ones((16,), jnp.float32)

        def orow(r, _):
            ones[r, :] = one16
            return 0
        lax.fori_loop(0, ch, orow, 0)

    pltpu.sync_copy(zeros_hbm.at[pl.ds(start, RPT)],
                    agg_sh.at[pl.ds(start, RPT)])
    if with_cnt:
        pltpu.sync_copy(zcnt_hbm.at[pl.ds(start, RPT)],
                        cnt_sh.at[pl.ds(start, RPT)])

    @pl.when(s == NS - 1)
    def _ztail():
        pltpu.sync_copy(zeros_hbm.at[pl.ds(TAIL, N - TAIL)],
                        agg_sh.at[pl.ds(TAIL, N - TAIL)])
        if with_cnt:
            pltpu.sync_copy(zcnt_hbm.at[pl.ds(TAIL, N - TAIL)],
                            cnt_sh.at[pl.ds(TAIL, N - TAIL)])

    plsc.subcore_barrier()

    ntotal = E // ch
    base = ntotal // NW
    rem = ntotal % NW
    cw = base + jnp.where(wid < rem, 1, 0)
    rbase = wid * base + jnp.minimum(wid, rem)

    def load_bulk(b):
        bb = b % 2
        pltpu.sync_copy(src_hbm.at[pl.ds(rbase + b * kb, kb)], srcb.at[bb])
        pltpu.sync_copy(dst_hbm.at[pl.ds(rbase + b * kb, kb)], dstb.at[bb])

    def fire(j):
        bb = (j // kb) % 2
        pltpu.async_copy(x_hbm.at[srcb.at[bb, j % kb]], rows.at[j % depth],
                         gsem.at[j % depth])

    def wait_scatter(j):
        pltpu.make_async_copy(rows.at[j % depth], agg_sh.at[pl.ds(0, ch)],
                              ssem.at[j % depth]).wait()
        if with_cnt:
            pltpu.make_async_copy(ones, cnt_sh.at[pl.ds(0, ch)],
                                  ssem.at[j % depth]).wait()

    load_bulk(0)
    fire(0)
    for k in range(1, depth - 1):
        @pl.when(k < cw)
        def _prefire(k=k):
            fire(k)

    def chunk(j, _):
        nj = j + depth - 1

        @pl.when(jnp.logical_and(nj % kb == 0, nj < cw))
        def _load():
            load_bulk(nj // kb)

        @pl.when(nj < cw)
        def _fire():
            @pl.when(nj >= depth)
            def _ws():
                wait_scatter(nj - depth)
            fire(nj)

        bb = (j // kb) % 2
        pltpu.make_async_copy(x_hbm.at[pl.ds(0, ch)], rows.at[j % depth],
                              gsem.at[j % depth]).wait()
        pltpu.async_copy(rows.at[j % depth], agg_sh.at[dstb.at[bb, j % kb]],
                         ssem.at[j % depth], add=True)
        if with_cnt:
            pltpu.async_copy(ones, cnt_sh.at[dstb.at[bb, j % kb]],
                             ssem.at[j % depth], add=True)
        return 0
    lax.fori_loop(0, cw, chunk, 0)

    for d in range(1, depth + 1):
        @pl.when(cw - d >= 0)
        def _drain(d=d):
            wait_scatter(cw - d)

    plsc.subcore_barrier()

    pltpu.sync_copy(agg_sh.at[pl.ds(start, RPT)],
                    aggp.at[c, pl.ds(start, RPT)])
    if with_cnt:
        pltpu.sync_copy(cnt_sh.at[pl.ds(start, RPT)],
                        cntp.at[c, pl.ds(start, RPT)])

    @pl.when(s == NS - 1)
    def _ctail():
        pltpu.sync_copy(agg_sh.at[pl.ds(TAIL, N - TAIL)],
                        aggp.at[c, pl.ds(TAIL, N - TAIL)])
        if with_cnt:
            pltpu.sync_copy(cnt_sh.at[pl.ds(TAIL, N - TAIL)],
                            cntp.at[c, pl.ds(TAIL, N - TAIL)])


@functools.cache
def _sc_agg(with_cnt: bool, ch: int, kb: int, depth: int):
    mesh = plsc.VectorSubcoreMesh(core_axis_name="c", subcore_axis_name="s",
                                  num_cores=NC, num_subcores=NS)
    out_type = [jax.ShapeDtypeStruct((NC, N, D), jnp.float32)]
    scratch = [
        pltpu.VMEM_SHARED((N, D), jnp.float32),
    ]
    if with_cnt:
        out_type.append(jax.ShapeDtypeStruct((NC, N, CNTW), jnp.float32))
        scratch.append(pltpu.VMEM_SHARED((N, CNTW), jnp.float32))
    scratch += [
        pltpu.VMEM((2, kb, ch), jnp.int32),
        pltpu.VMEM((2, kb, ch), jnp.int32),
        pltpu.VMEM((depth, ch, D), jnp.float32),
    ]
    if with_cnt:
        scratch.append(pltpu.VMEM((ch, CNTW), jnp.float32))
    scratch.append(pltpu.SemaphoreType.DMA((depth,)))
    scratch.append(pltpu.SemaphoreType.DMA((depth,)))

    if with_cnt:
        def body(x_hbm, src_hbm, dst_hbm, zeros_hbm, zcnt_hbm, aggp, cntp,
                 agg_sh, cnt_sh, srcb, dstb, rows, ones, gsem, ssem):
            _sc_common(True, ch, kb, depth,
                       x_hbm, src_hbm, dst_hbm, zeros_hbm, zcnt_hbm, aggp,
                       cntp, agg_sh, cnt_sh, srcb, dstb, rows, ones, gsem,
                       ssem)
    else:
        def body(x_hbm, src_hbm, dst_hbm, zeros_hbm, aggp,
                 agg_sh, srcb, dstb, rows, gsem, ssem):
            _sc_common(False, ch, kb, depth,
                       x_hbm, src_hbm, dst_hbm, zeros_hbm, None, aggp, None,
                       agg_sh, None, srcb, dstb, rows, None, gsem, ssem)

    return pl.kernel(body, out_type=tuple(out_type), mesh=mesh,
                     scratch_types=tuple(scratch),
                     compiler_params=pltpu.CompilerParams(
                         use_tc_tiling_on_sc=False))


def _dot_t(a, w):
    return lax.dot_general(a, w, (((1,), (1,)), ((), ())),
                           preferred_element_type=jnp.float32)


def _tc_xr_body(x_ref, wr_ref, o_ref):
    o_ref[...] = _dot_t(x_ref[...], wr_ref[...])


def _tc_layer_body(act, aggp_ref, cntp_ref, xr_ref, wl_ref, bl_ref, o_ref):
    agg = aggp_ref[0] + aggp_ref[1]
    cnt = cntp_ref[0, :, 0:1] + cntp_ref[1, :, 0:1]
    mean = agg / jnp.maximum(cnt, 1.0)
    out = _dot_t(mean, wl_ref[...]) + bl_ref[...] + xr_ref[...]
    if act == "elu":
        o_ref[...] = jnp.where(out > 0, out,
                               jnp.exp(jnp.minimum(out, 0.0)) - 1.0)
    else:
        m = jnp.max(out, axis=1, keepdims=True)
        lse = jnp.log(jnp.sum(jnp.exp(out - m), axis=1, keepdims=True)) + m
        o_ref[...] = out - lse


BR = 1000


@functools.cache
def _tc_xr():
    return pl.pallas_call(
        _tc_xr_body,
        grid=(N // BR,),
        in_specs=[
            pl.BlockSpec((BR, D), lambda i: (i, 0)),
            pl.BlockSpec((D, D), lambda i: (0, 0)),
        ],
        out_specs=pl.BlockSpec((BR, D), lambda i: (i, 0)),
        out_shape=jax.ShapeDtypeStruct((N, D), jnp.float32),
    )


@functools.cache
def _tc_layer(act: str):
    return pl.pallas_call(
        functools.partial(_tc_layer_body, act),
        grid=(N // BR,),
        in_specs=[
            pl.BlockSpec((NC, BR, D), lambda i: (0, i, 0)),
            pl.BlockSpec((NC, BR, CNTW), lambda i: (0, i, 0)),
            pl.BlockSpec((BR, D), lambda i: (i, 0)),
            pl.BlockSpec((D, D), lambda i: (0, 0)),
            pl.BlockSpec((1, D), lambda i: (0, 0)),
        ],
        out_specs=pl.BlockSpec((BR, D), lambda i: (i, 0)),
        out_shape=jax.ShapeDtypeStruct((N, D), jnp.float32),
    )


@jax.jit
def kernel(x, edge_index, W1l, b1l, W1r, W2l, b2l, W2r):
    src = edge_index[0].astype(jnp.int32)
    dst = edge_index[1].astype(jnp.int32)
    src1 = src.reshape(E // CH1, CH1)
    dst1 = dst.reshape(E // CH1, CH1)
    src2 = src.reshape(E // CH2, CH2)
    dst2 = dst.reshape(E // CH2, CH2)
    zeros = jnp.zeros((N, D), jnp.float32)
    zcnt = jnp.zeros((N, CNTW), jnp.float32)
    xr1 = _tc_xr()(x, W1r)
    aggp1, cntp = _sc_agg(True, CH1, KB1, DEPTH1)(x, src1, dst1, zeros, zcnt)
    h = _tc_layer("elu")(aggp1, cntp, xr1, W1l, b1l.reshape(1, D))
    xr2 = _tc_xr()(h, W2r)
    aggp2, = _sc_agg(False, CH2, KB2, DEPTH2)(h, src2, dst2, zeros)
    return _tc_layer("lsm")(aggp2, cntp, xr2, W2l, b2l.reshape(1, D))


# speedup vs baseline: 1.0589x; 1.0246x over previous
import functools

import jax
import jax.numpy as jnp
from jax import lax
from jax.experimental import pallas as pl
from jax.experimental.pallas import tpu as pltpu
from jax.experimental.pallas import tpu_sc as plsc

NC = 2
NS = 16
NW = NC * NS
N = 10000
D = 128
E = 320000
RPT = 624
TAIL = NS * RPT
CNTW = 16

CH1, KB1, DEPTH1 = 128, 6, 2
CH2, KB2, DEPTH2 = 128, 3, 3


def _sc_common(with_cnt, ch, kb, depth,
               x_hbm, src_hbm, dst_hbm, zeros_hbm, zcnt_hbm, aggp, cntp,
               agg_sh, cnt_sh, srcb, dstb, rows, ones, gsem, ssem):
    c = lax.axis_index("c")
    s = lax.axis_index("s")
    wid = c * NS + s
    start = s * RPT

    if with_cnt:
        one16 = jnp.ones((16,), jnp.float32)

        def orow(r, _):
            ones[r, :] = one16
            return 0
        lax.fori_loop(0, ch, orow, 0)

    pltpu.sync_copy(zeros_hbm.at[pl.ds(start, RPT)],
                    agg_sh.at[pl.ds(start, RPT)])
    if with_cnt:
        pltpu.sync_copy(zcnt_hbm.at[pl.ds(start, RPT)],
                        cnt_sh.at[pl.ds(start, RPT)])

    @pl.when(s == NS - 1)
    def _ztail():
        pltpu.sync_copy(zeros_hbm.at[pl.ds(TAIL, N - TAIL)],
                        agg_sh.at[pl.ds(TAIL, N - TAIL)])
        if with_cnt:
            pltpu.sync_copy(zcnt_hbm.at[pl.ds(TAIL, N - TAIL)],
                            cnt_sh.at[pl.ds(TAIL, N - TAIL)])

    plsc.subcore_barrier()

    ntotal = E // ch
    base = ntotal // NW
    rem = ntotal % NW
    cw = base + jnp.where(wid < rem, 1, 0)
    rbase = wid * base + jnp.minimum(wid, rem)

    def load_bulk(b):
        bb = b % 2
        pltpu.sync_copy(src_hbm.at[pl.ds(rbase + b * kb, kb)], srcb.at[bb])
        pltpu.sync_copy(dst_hbm.at[pl.ds(rbase + b * kb, kb)], dstb.at[bb])

    def fire(j):
        bb = (j // kb) % 2
        pltpu.async_copy(x_hbm.at[srcb.at[bb, j % kb]], rows.at[j % depth],
                         gsem.at[j % depth])

    def wait_scatter(j):
        pltpu.make_async_copy(rows.at[j % depth], agg_sh.at[pl.ds(0, ch)],
                              ssem.at[j % depth]).wait()
        if with_cnt:
            pltpu.make_async_copy(ones, cnt_sh.at[pl.ds(0, ch)],
                                  ssem.at[j % depth]).wait()

    load_bulk(0)
    fire(0)
    for k in range(1, depth - 1):
        @pl.when(k < cw)
        def _prefire(k=k):
            fire(k)

    def chunk(j, _):
        nj = j + depth - 1

        @pl.when(jnp.logical_and(nj % kb == 0, nj < cw))
        def _load():
            load_bulk(nj // kb)

        @pl.when(nj < cw)
        def _fire():
            @pl.when(nj >= depth)
            def _ws():
                wait_scatter(nj - depth)
            fire(nj)

        bb = (j // kb) % 2
        pltpu.make_async_copy(x_hbm.at[pl.ds(0, ch)], rows.at[j % depth],
                              gsem.at[j % depth]).wait()
        pltpu.async_copy(rows.at[j % depth], agg_sh.at[dstb.at[bb, j % kb]],
                         ssem.at[j % depth], add=True)
        if with_cnt:
            pltpu.async_copy(ones, cnt_sh.at[dstb.at[bb, j % kb]],
                             ssem.at[j % depth], add=True)
        return 0
    lax.fori_loop(0, cw, chunk, 0)

    for d in range(1, depth + 1):
        @pl.when(cw - d >= 0)
        def _drain(d=d):
            wait_scatter(cw - d)

    plsc.subcore_barrier()

    pltpu.sync_copy(agg_sh.at[pl.ds(start, RPT)],
                    aggp.at[c, pl.ds(start, RPT)])
    if with_cnt:
        pltpu.sync_copy(cnt_sh.at[pl.ds(start, RPT)],
                        cntp.at[c, pl.ds(start, RPT)])

    @pl.when(s == NS - 1)
    def _ctail():
        pltpu.sync_copy(agg_sh.at[pl.ds(TAIL, N - TAIL)],
                        aggp.at[c, pl.ds(TAIL, N - TAIL)])
        if with_cnt:
            pltpu.sync_copy(cnt_sh.at[pl.ds(TAIL, N - TAIL)],
                            cntp.at[c, pl.ds(TAIL, N - TAIL)])


@functools.cache
def _sc_agg(with_cnt: bool, ch: int, kb: int, depth: int):
    mesh = plsc.VectorSubcoreMesh(core_axis_name="c", subcore_axis_name="s",
                                  num_cores=NC, num_subcores=NS)
    out_type = [jax.ShapeDtypeStruct((NC, N, D), jnp.float32)]
    scratch = [
        pltpu.VMEM_SHARED((N, D), jnp.float32),
    ]
    if with_cnt:
        out_type.append(jax.ShapeDtypeStruct((NC, N, CNTW), jnp.float32))
        scratch.append(pltpu.VMEM_SHARED((N, CNTW), jnp.float32))
    scratch += [
        pltpu.VMEM((2, kb, ch), jnp.int32),
        pltpu.VMEM((2, kb, ch), jnp.int32),
        pltpu.VMEM((depth, ch, D), jnp.float32),
    ]
    if with_cnt:
        scratch.append(pltpu.VMEM((ch, CNTW), jnp.float32))
    scratch.append(pltpu.SemaphoreType.DMA((depth,)))
    scratch.append(pltpu.SemaphoreType.DMA((depth,)))

    if with_cnt:
        def body(x_hbm, src_hbm, dst_hbm, zeros_hbm, zcnt_hbm, aggp, cntp,
                 agg_sh, cnt_sh, srcb, dstb, rows, ones, gsem, ssem):
            _sc_common(True, ch, kb, depth,
                       x_hbm, src_hbm, dst_hbm, zeros_hbm, zcnt_hbm, aggp,
                       cntp, agg_sh, cnt_sh, srcb, dstb, rows, ones, gsem,
                       ssem)
    else:
        def body(x_hbm, src_hbm, dst_hbm, zeros_hbm, aggp,
                 agg_sh, srcb, dstb, rows, gsem, ssem):
            _sc_common(False, ch, kb, depth,
                       x_hbm, src_hbm, dst_hbm, zeros_hbm, None, aggp, None,
                       agg_sh, None, srcb, dstb, rows, None, gsem, ssem)

    return pl.kernel(body, out_type=tuple(out_type), mesh=mesh,
                     scratch_types=tuple(scratch),
                     compiler_params=pltpu.CompilerParams(
                         use_tc_tiling_on_sc=False))


def _dot_t(a, w):
    return lax.dot_general(a, w, (((1,), (1,)), ((), ())),
                           preferred_element_type=jnp.float32)


def _tc_xr_body(x_ref, wr_ref, o_ref):
    o_ref[...] = _dot_t(x_ref[...], wr_ref[...])


def _tc_layer_body(act, aggp_ref, cntp_ref, xr_ref, wl_ref, bl_ref, o_ref):
    agg = aggp_ref[0] + aggp_ref[1]
    cnt = cntp_ref[0, :, 0:1] + cntp_ref[1, :, 0:1]
    mean = agg / jnp.maximum(cnt, 1.0)
    out = _dot_t(mean, wl_ref[...]) + bl_ref[...] + xr_ref[...]
    if act == "elu":
        o_ref[...] = jnp.where(out > 0, out,
                               jnp.exp(jnp.minimum(out, 0.0)) - 1.0)
    else:
        m = jnp.max(out, axis=1, keepdims=True)
        lse = jnp.log(jnp.sum(jnp.exp(out - m), axis=1, keepdims=True)) + m
        o_ref[...] = out - lse


BR = 1000


@functools.cache
def _tc_xr():
    return pl.pallas_call(
        _tc_xr_body,
        grid=(N // BR,),
        in_specs=[
            pl.BlockSpec((BR, D), lambda i: (i, 0)),
            pl.BlockSpec((D, D), lambda i: (0, 0)),
        ],
        out_specs=pl.BlockSpec((BR, D), lambda i: (i, 0)),
        out_shape=jax.ShapeDtypeStruct((N, D), jnp.float32),
    )


@functools.cache
def _tc_layer(act: str):
    return pl.pallas_call(
        functools.partial(_tc_layer_body, act),
        grid=(N // BR,),
        in_specs=[
            pl.BlockSpec((NC, BR, D), lambda i: (0, i, 0)),
            pl.BlockSpec((NC, BR, CNTW), lambda i: (0, i, 0)),
            pl.BlockSpec((BR, D), lambda i: (i, 0)),
            pl.BlockSpec((D, D), lambda i: (0, 0)),
            pl.BlockSpec((1, D), lambda i: (0, 0)),
        ],
        out_specs=pl.BlockSpec((BR, D), lambda i: (i, 0)),
        out_shape=jax.ShapeDtypeStruct((N, D), jnp.float32),
    )


@jax.jit
def kernel(x, edge_index, W1l, b1l, W1r, W2l, b2l, W2r):
    src = edge_index[0].astype(jnp.int32)
    dst = edge_index[1].astype(jnp.int32)
    src1 = src.reshape(E // CH1, CH1)
    dst1 = dst.reshape(E // CH1, CH1)
    src2 = src.reshape(E // CH2, CH2)
    dst2 = dst.reshape(E // CH2, CH2)
    zeros = jnp.zeros((N, D), jnp.float32)
    zcnt = jnp.zeros((N, CNTW), jnp.float32)
    xr1 = _tc_xr()(x, W1r)
    aggp1, cntp = _sc_agg(True, CH1, KB1, DEPTH1)(x, src1, dst1, zeros, zcnt)
    h = _tc_layer("elu")(aggp1, cntp, xr1, W1l, b1l.reshape(1, D))
    xr2 = _tc_xr()(h, W2r)
    aggp2, = _sc_agg(False, CH2, KB2, DEPTH2)(h, src2, dst2, zeros)
    return _tc_layer("lsm")(aggp2, cntp, xr2, W2l, b2l.reshape(1, D))


# speedup vs baseline: 1.0872x; 1.0267x over previous
import functools

import jax
import jax.numpy as jnp
from jax import lax
from jax.experimental import pallas as pl
from jax.experimental.pallas import tpu as pltpu
from jax.experimental.pallas import tpu_sc as plsc

NC = 2
NS = 16
NW = NC * NS
N = 10000
D = 128
E = 320000
RPT = 624
TAIL = NS * RPT
CNTW = 16

CH1, KB1, DEPTH1 = 80, 5, 3
CH2, KB2, DEPTH2 = 128, 3, 3


def _sc_common(with_cnt, ch, kb, depth,
               x_hbm, src_hbm, dst_hbm, zeros_hbm, zcnt_hbm, aggp, cntp,
               agg_sh, cnt_sh, srcb, dstb, rows, ones, gsem, ssem):
    c = lax.axis_index("c")
    s = lax.axis_index("s")
    wid = c * NS + s
    start = s * RPT

    if with_cnt:
        one16 = jnp.ones((16,), jnp.float32)

        def orow(r, _):
            ones[r, :] = one16
            return 0
        lax.fori_loop(0, ch, orow, 0)

    pltpu.sync_copy(zeros_hbm.at[pl.ds(start, RPT)],
                    agg_sh.at[pl.ds(start, RPT)])
    if with_cnt:
        pltpu.sync_copy(zcnt_hbm.at[pl.ds(start, RPT)],
                        cnt_sh.at[pl.ds(start, RPT)])

    @pl.when(s == NS - 1)
    def _ztail():
        pltpu.sync_copy(zeros_hbm.at[pl.ds(TAIL, N - TAIL)],
                        agg_sh.at[pl.ds(TAIL, N - TAIL)])
        if with_cnt:
            pltpu.sync_copy(zcnt_hbm.at[pl.ds(TAIL, N - TAIL)],
                            cnt_sh.at[pl.ds(TAIL, N - TAIL)])

    plsc.subcore_barrier()

    ntotal = E // ch
    base = ntotal // NW
    rem = ntotal % NW
    cw = base + jnp.where(wid < rem, 1, 0)
    rbase = wid * base + jnp.minimum(wid, rem)

    def load_bulk(b):
        bb = b % 2
        pltpu.sync_copy(src_hbm.at[pl.ds(rbase + b * kb, kb)], srcb.at[bb])
        pltpu.sync_copy(dst_hbm.at[pl.ds(rbase + b * kb, kb)], dstb.at[bb])

    def fire(j):
        bb = (j // kb) % 2
        pltpu.async_copy(x_hbm.at[srcb.at[bb, j % kb]], rows.at[j % depth],
                         gsem.at[j % depth])

    def wait_scatter(j):
        pltpu.make_async_copy(rows.at[j % depth], agg_sh.at[pl.ds(0, ch)],
                              ssem.at[j % depth]).wait()
        if with_cnt:
            pltpu.make_async_copy(ones, cnt_sh.at[pl.ds(0, ch)],
                                  ssem.at[j % depth]).wait()

    load_bulk(0)
    fire(0)
    for k in range(1, depth - 1):
        @pl.when(k < cw)
        def _prefire(k=k):
            fire(k)

    def chunk(j, _):
        nj = j + depth - 1

        @pl.when(jnp.logical_and(nj % kb == 0, nj < cw))
        def _load():
            load_bulk(nj // kb)

        @pl.when(nj < cw)
        def _fire():
            @pl.when(nj >= depth)
            def _ws():
                wait_scatter(nj - depth)
            fire(nj)

        bb = (j // kb) % 2
        pltpu.make_async_copy(x_hbm.at[pl.ds(0, ch)], rows.at[j % depth],
                              gsem.at[j % depth]).wait()
        pltpu.async_copy(rows.at[j % depth], agg_sh.at[dstb.at[bb, j % kb]],
                         ssem.at[j % depth], add=True)
        if with_cnt:
            pltpu.async_copy(ones, cnt_sh.at[dstb.at[bb, j % kb]],
                             ssem.at[j % depth], add=True)
        return 0
    lax.fori_loop(0, cw, chunk, 0)

    for d in range(1, depth + 1):
        @pl.when(cw - d >= 0)
        def _drain(d=d):
            wait_scatter(cw - d)

    plsc.subcore_barrier()

    pltpu.sync_copy(agg_sh.at[pl.ds(start, RPT)],
                    aggp.at[c, pl.ds(start, RPT)])
    if with_cnt:
        pltpu.sync_copy(cnt_sh.at[pl.ds(start, RPT)],
                        cntp.at[c, pl.ds(start, RPT)])

    @pl.when(s == NS - 1)
    def _ctail():
        pltpu.sync_copy(agg_sh.at[pl.ds(TAIL, N - TAIL)],
                        aggp.at[c, pl.ds(TAIL, N - TAIL)])
        if with_cnt:
            pltpu.sync_copy(cnt_sh.at[pl.ds(TAIL, N - TAIL)],
                            cntp.at[c, pl.ds(TAIL, N - TAIL)])


@functools.cache
def _sc_agg(with_cnt: bool, ch: int, kb: int, depth: int):
    mesh = plsc.VectorSubcoreMesh(core_axis_name="c", subcore_axis_name="s",
                                  num_cores=NC, num_subcores=NS)
    out_type = [jax.ShapeDtypeStruct((NC, N, D), jnp.float32)]
    scratch = [
        pltpu.VMEM_SHARED((N, D), jnp.float32),
    ]
    if with_cnt:
        out_type.append(jax.ShapeDtypeStruct((NC, N, CNTW), jnp.float32))
        scratch.append(pltpu.VMEM_SHARED((N, CNTW), jnp.float32))
    scratch += [
        pltpu.VMEM((2, kb, ch), jnp.int32),
        pltpu.VMEM((2, kb, ch), jnp.int32),
        pltpu.VMEM((depth, ch, D), jnp.float32),
    ]
    if with_cnt:
        scratch.append(pltpu.VMEM((ch, CNTW), jnp.float32))
    scratch.append(pltpu.SemaphoreType.DMA((depth,)))
    scratch.append(pltpu.SemaphoreType.DMA((depth,)))

    if with_cnt:
        def body(x_hbm, src_hbm, dst_hbm, zeros_hbm, zcnt_hbm, aggp, cntp,
                 agg_sh, cnt_sh, srcb, dstb, rows, ones, gsem, ssem):
            _sc_common(True, ch, kb, depth,
                       x_hbm, src_hbm, dst_hbm, zeros_hbm, zcnt_hbm, aggp,
                       cntp, agg_sh, cnt_sh, srcb, dstb, rows, ones, gsem,
                       ssem)
    else:
        def body(x_hbm, src_hbm, dst_hbm, zeros_hbm, aggp,
                 agg_sh, srcb, dstb, rows, gsem, ssem):
            _sc_common(False, ch, kb, depth,
                       x_hbm, src_hbm, dst_hbm, zeros_hbm, None, aggp, None,
                       agg_sh, None, srcb, dstb, rows, None, gsem, ssem)

    return pl.kernel(body, out_type=tuple(out_type), mesh=mesh,
                     scratch_types=tuple(scratch),
                     compiler_params=pltpu.CompilerParams(
                         use_tc_tiling_on_sc=False))


def _dot_t(a, w):
    return lax.dot_general(a, w, (((1,), (1,)), ((), ())),
                           preferred_element_type=jnp.float32)


def _tc_xr_body(x_ref, wr_ref, o_ref):
    o_ref[...] = _dot_t(x_ref[...], wr_ref[...])


def _tc_layer_body(act, aggp_ref, cntp_ref, xr_ref, wl_ref, bl_ref, o_ref):
    agg = aggp_ref[0] + aggp_ref[1]
    cnt = cntp_ref[0, :, 0:1] + cntp_ref[1, :, 0:1]
    mean = agg / jnp.maximum(cnt, 1.0)
    out = _dot_t(mean, wl_ref[...]) + bl_ref[...] + xr_ref[...]
    if act == "elu":
        o_ref[...] = jnp.where(out > 0, out,
                               jnp.exp(jnp.minimum(out, 0.0)) - 1.0)
    else:
        m = jnp.max(out, axis=1, keepdims=True)
        lse = jnp.log(jnp.sum(jnp.exp(out - m), axis=1, keepdims=True)) + m
        o_ref[...] = out - lse


BR = 1000


@functools.cache
def _tc_xr():
    return pl.pallas_call(
        _tc_xr_body,
        grid=(N // BR,),
        in_specs=[
            pl.BlockSpec((BR, D), lambda i: (i, 0)),
            pl.BlockSpec((D, D), lambda i: (0, 0)),
        ],
        out_specs=pl.BlockSpec((BR, D), lambda i: (i, 0)),
        out_shape=jax.ShapeDtypeStruct((N, D), jnp.float32),
    )


@functools.cache
def _tc_layer(act: str):
    return pl.pallas_call(
        functools.partial(_tc_layer_body, act),
        grid=(N // BR,),
        in_specs=[
            pl.BlockSpec((NC, BR, D), lambda i: (0, i, 0)),
            pl.BlockSpec((NC, BR, CNTW), lambda i: (0, i, 0)),
            pl.BlockSpec((BR, D), lambda i: (i, 0)),
            pl.BlockSpec((D, D), lambda i: (0, 0)),
            pl.BlockSpec((1, D), lambda i: (0, 0)),
        ],
        out_specs=pl.BlockSpec((BR, D), lambda i: (i, 0)),
        out_shape=jax.ShapeDtypeStruct((N, D), jnp.float32),
    )


@jax.jit
def kernel(x, edge_index, W1l, b1l, W1r, W2l, b2l, W2r):
    src = edge_index[0].astype(jnp.int32)
    dst = edge_index[1].astype(jnp.int32)
    src1 = src.reshape(E // CH1, CH1)
    dst1 = dst.reshape(E // CH1, CH1)
    src2 = src.reshape(E // CH2, CH2)
    dst2 = dst.reshape(E // CH2, CH2)
    zeros = jnp.zeros((N, D), jnp.float32)
    zcnt = jnp.zeros((N, CNTW), jnp.float32)
    xr1 = _tc_xr()(x, W1r)
    aggp1, cntp = _sc_agg(True, CH1, KB1, DEPTH1)(x, src1, dst1, zeros, zcnt)
    h = _tc_layer("elu")(aggp1, cntp, xr1, W1l, b1l.reshape(1, D))
    xr2 = _tc_xr()(h, W2r)
    aggp2, = _sc_agg(False, CH2, KB2, DEPTH2)(h, src2, dst2, zeros)
    return _tc_layer("lsm")(aggp2, cntp, xr2, W2l, b2l.reshape(1, D))
